# baseline (speedup 1.0000x reference)
import functools

import jax
import jax.numpy as jnp
from jax import lax
from jax.experimental import pallas as pl
from jax.experimental.pallas import tpu as pltpu
from jax.experimental.pallas import tpu_sc as plsc

N_NODES = 10000
N_EDGES = 160000
E_PAD = 163840
IN_DIM = 16
OUT_DIM = 16
E_DIM = 6
EDGE_H = 64
AUG = 32

NW = 32
EDGES_PER_W = E_PAD // NW
G_EDGES = 1024
N_GROUPS = EDGES_PER_W // G_EDGES
CH = 128
NCH = EDGES_PER_W // CH
N_PAD = 10240
ROWS_PER_TILE = N_PAD // 16

HS_ROWS = E_PAD * IN_DIM // 128
HSR_PER_W = HS_ROWS // NW
MSG_ROWS = E_PAD * AUG // 128
MSGR_PER_W = MSG_ROWS // NW

BE = 2048
GRID = E_PAD // BE
RB = BE // 8


@functools.cache
def _sc_mesh():
    return plsc.VectorSubcoreMesh(core_axis_name="c", subcore_axis_name="s",
                                  num_cores=2)


@functools.cache
def _gather_hsrc_kernel():
    @functools.partial(
        pl.kernel,
        mesh=_sc_mesh(),
        compiler_params=pltpu.CompilerParams(use_tc_tiling_on_sc=False),
        out_type=jax.ShapeDtypeStruct((HS_ROWS, 128), jnp.float32),
        scratch_types=[
            pltpu.VMEM((N_GROUPS, 8, CH), jnp.int32),
            pltpu.VMEM((2, G_EDGES, IN_DIM), jnp.float32),
            pltpu.SemaphoreType.DMA,
            pltpu.SemaphoreType.DMA,
        ],
    )
    def _gather_hsrc(h_hbm, src_hbm, out_hbm, idx_v, rows_v, sem_g, sem_w):
        c = lax.axis_index("c")
        s = lax.axis_index("s")
        w = c * 16 + s
        for g in range(N_GROUPS):
            pltpu.sync_copy(src_hbm.at[w + NW * g], idx_v.at[g])

        def fire(g, b):
            for p in range(8):
                pltpu.async_copy(
                    h_hbm.at[idx_v.at[g, p]],
                    rows_v.at[b, pl.ds(p * CH, CH)], sem_g)

        def drain(g, b):
            for p in range(8):
                pltpu.make_async_copy(
                    h_hbm.at[idx_v.at[g, p]],
                    rows_v.at[b, pl.ds(p * CH, CH)], sem_g).wait()

        def wb(g, b, do_wait):
            for p in range(8):
                a = (rows_v.at[b, pl.ds(p * CH, CH)],
                     out_hbm.at[pl.ds((w + NW * g) * CH, CH),
                                pl.ds(p * IN_DIM, IN_DIM)])
                if do_wait:
                    pltpu.make_async_copy(*a, sem_w).wait()
                else:
                    pltpu.async_copy(*a, sem_w)

        fire(0, 0)
        for g in range(N_GROUPS):
            b = g % 2
            drain(g, b)
            if g >= 2:
                wb(g - 2, b, True)
            if g + 1 < N_GROUPS:
                fire(g + 1, (g + 1) % 2)
            wb(g, b, False)
        for g in (N_GROUPS - 2, N_GROUPS - 1):
            wb(g, g % 2, True)

    return _gather_hsrc


@functools.cache
def _scatter_agg_kernel():
    @functools.partial(
        pl.kernel,
        mesh=_sc_mesh(),
        compiler_params=pltpu.CompilerParams(use_tc_tiling_on_sc=False),
        out_type=jax.ShapeDtypeStruct((2, N_PAD, AUG), jnp.float32),
        scratch_types=[
            pltpu.VMEM((N_GROUPS, 8, CH), jnp.int32),
            pltpu.VMEM((2, G_EDGES, AUG), jnp.float32),
            pltpu.VMEM_SHARED((N_PAD, AUG), jnp.float32),
            pltpu.SemaphoreType.DMA,
            pltpu.SemaphoreType.DMA,
        ],
    )
    def _scatter(msg_hbm, dst_hbm, zero_hbm, out_hbm, idx_v, rows_v, agg_sh,
                 sem_r, sem_s):
        c = lax.axis_index("c")
        s = lax.axis_index("s")
        w = c * 16 + s
        pltpu.sync_copy(zero_hbm.at[pl.ds(s * ROWS_PER_TILE, ROWS_PER_TILE)],
                        agg_sh.at[pl.ds(s * ROWS_PER_TILE, ROWS_PER_TILE)])
        for g in range(N_GROUPS):
            pltpu.sync_copy(dst_hbm.at[w + NW * g], idx_v.at[g])
        plsc.subcore_barrier()

        def read(g, b, do_wait):
            for q in range(4):
                a = (msg_hbm.at[pl.ds((w + NW * g) * 256, 256),
                                pl.ds(q * AUG, AUG)],
                     rows_v.at[b, pl.ds(q * 256, 256)])
                if do_wait:
                    pltpu.make_async_copy(*a, sem_r).wait()
                else:
                    pltpu.async_copy(*a, sem_r)

        def scatters(g, b, do_wait):
            for cch in range(8):
                a = (rows_v.at[b, pl.ds(cch * CH, CH)],
                     agg_sh.at[idx_v.at[g, cch]])
                if do_wait:
                    pltpu.make_async_copy(*a, sem_s).wait()
                else:
                    pltpu.async_copy(*a, sem_s, add=True)

        read(0, 0, False)
        for g in range(N_GROUPS):
            b = g % 2
            read(g, b, True)
            if g >= 2:
                scatters(g - 2, b, True)
            if g + 1 < N_GROUPS:
                read(g + 1, (g + 1) % 2, False)
            scatters(g, b, False)
        for g in (N_GROUPS - 2, N_GROUPS - 1):
            scatters(g, g % 2, True)
        plsc.subcore_barrier()
        pltpu.sync_copy(agg_sh.at[pl.ds(s * ROWS_PER_TILE, ROWS_PER_TILE)],
                        out_hbm.at[c, pl.ds(s * ROWS_PER_TILE, ROWS_PER_TILE)])

    return _scatter


def _dense_body(e_ref, hs_ref, w1_ref, b1_ref, w2_ref, b2_ref, r_ref, s_ref,
                out_ref):
    et = e_ref[...]
    hid = jnp.maximum(
        lax.dot_general(et, w1_ref[...], (((0,), (0,)), ((), ())),
                        preferred_element_type=jnp.float32)
        + b1_ref[...], 0.0)
    ew = (jnp.dot(hid, w2_ref[...], preferred_element_type=jnp.float32)
          + b2_ref[...])
    hs128 = hs_ref[...]
    i_blk = pl.program_id(0)
    rowid = lax.broadcasted_iota(jnp.int32, (RB, 1), 0)
    colid = lax.broadcasted_iota(jnp.int32, (RB, AUG - OUT_DIM), 1)
    onepad = jnp.where(colid == 0, 1.0, 0.0)
    augs = []
    for p in range(8):
        hs_p = hs128[:, p * IN_DIM:(p + 1) * IN_DIM]
        ew_p = ew[p * RB:(p + 1) * RB]
        hrep = jnp.dot(hs_p, r_ref[...], preferred_element_type=jnp.float32)
        msg = jnp.dot(hrep * ew_p, s_ref[...],
                      preferred_element_type=jnp.float32)
        augs.append(jnp.concatenate([msg, onepad], axis=1))
    top = jnp.concatenate(augs[0:4], axis=1)
    bot = jnp.concatenate(augs[4:8], axis=1)
    full = jnp.concatenate([top, bot], axis=0)

    n_full = (N_EDGES // BE)

    @pl.when(i_blk < n_full)
    def _store_fast():
        out_ref[...] = full

    @pl.when(i_blk >= n_full)
    def _store_masked():
        lane = lax.broadcasted_iota(jnp.int32, (1, 128), 1)
        base = N_EDGES - BE * i_blk - RB * (lane // AUG)
        m_top = jnp.broadcast_to(rowid < base, (RB, 128))
        m_bot = jnp.broadcast_to(rowid < (base - 4 * RB), (RB, 128))
        mfull = jnp.concatenate([m_top, m_bot], axis=0)
        out_ref[...] = jnp.where(mfull, full, 0.0)


def _dense_msg(e_perm, hs_pk, W1, b1, W2, b2, R, S):
    return pl.pallas_call(
        _dense_body,
        grid=(GRID,),
        in_specs=[
            pl.BlockSpec((E_DIM, BE), lambda i: (0, i)),
            pl.BlockSpec((RB, 128), lambda i: (i, 0)),
            pl.BlockSpec((E_DIM, EDGE_H), lambda i: (0, 0)),
            pl.BlockSpec((1, EDGE_H), lambda i: (0, 0)),
            pl.BlockSpec((EDGE_H, IN_DIM * OUT_DIM), lambda i: (0, 0)),
            pl.BlockSpec((1, IN_DIM * OUT_DIM), lambda i: (0, 0)),
            pl.BlockSpec((IN_DIM, IN_DIM * OUT_DIM), lambda i: (0, 0)),
            pl.BlockSpec((IN_DIM * OUT_DIM, OUT_DIM), lambda i: (0, 0)),
        ],
        out_specs=pl.BlockSpec((2 * RB, 128), lambda i: (i, 0)),
        out_shape=jax.ShapeDtypeStruct((MSG_ROWS, 128), jnp.float32),
    )(e_perm, hs_pk, W1, b1, W2, b2, R, S)


def _final_body(parts_ref, bias_ref, gamma_ref, beta_ref, out_ref):
    p = parts_ref[0, 0:N_NODES] + parts_ref[1, 0:N_NODES]
    agg = p[:, 0:OUT_DIM]
    deg = p[:, OUT_DIM:OUT_DIM + 1]
    rst = agg / jnp.maximum(deg, 1.0) + bias_ref[...]
    rst = jnp.maximum(rst, 0.0)
    mean = jnp.mean(rst, axis=0, keepdims=True)
    var = jnp.mean((rst - mean) * (rst - mean), axis=0, keepdims=True)
    out_ref[...] = ((rst - mean) * lax.rsqrt(var + 1e-5) * gamma_ref[...]
                    + beta_ref[...])


def _finalize(parts, nn_bias, gamma, beta):
    return pl.pallas_call(
        _final_body,
        in_specs=[
            pl.BlockSpec((2, N_PAD, AUG), lambda: (0, 0, 0)),
            pl.BlockSpec((1, OUT_DIM), lambda: (0, 0)),
            pl.BlockSpec((1, OUT_DIM), lambda: (0, 0)),
            pl.BlockSpec((1, OUT_DIM), lambda: (0, 0)),
        ],
        out_specs=pl.BlockSpec((N_NODES, OUT_DIM), lambda: (0, 0)),
        out_shape=jax.ShapeDtypeStruct((N_NODES, OUT_DIM), jnp.float32),
    )(parts, nn_bias, gamma, beta)


def kernel(h, e, edge_index, W1, b1, W2, b2, nn_bias, gamma, beta):
    pad = E_PAD - N_EDGES
    src = jnp.pad(edge_index[0], (0, pad))
    dst = jnp.pad(edge_index[1], (0, pad))
    e_t = jnp.pad(e.T, ((0, 0), (0, pad)))

    src_sc = (src.reshape(GRID, 8, 2, CH)
              .transpose(0, 2, 1, 3).reshape(2 * GRID, 8, CH))
    dst_sc = dst.reshape(2 * GRID, 8, CH)

    hs_pk = _gather_hsrc_kernel()(h, src_sc)

    eye = jnp.eye(OUT_DIM, dtype=jnp.float32)
    R = jnp.kron(jnp.eye(IN_DIM, dtype=jnp.float32),
                 jnp.ones((1, OUT_DIM), jnp.float32))
    S = jnp.tile(eye, (IN_DIM, 1))
    msg = _dense_msg(e_t, hs_pk, W1, b1.reshape(1, EDGE_H), W2,
                     b2.reshape(1, IN_DIM * OUT_DIM), R, S)

    zero = jnp.zeros((N_PAD, AUG), jnp.float32)
    parts = _scatter_agg_kernel()(msg, dst_sc, zero)

    return _finalize(parts, nn_bias.reshape(1, OUT_DIM),
                     gamma.reshape(1, OUT_DIM), beta.reshape(1, OUT_DIM))


# speedup vs baseline: 1.0312x; 1.0312x over previous
import functools

import jax
import jax.numpy as jnp
from jax import lax
from jax.experimental import pallas as pl
from jax.experimental.pallas import tpu as pltpu
from jax.experimental.pallas import tpu_sc as plsc

N_NODES = 10000
N_EDGES = 160000
E_PAD = 163840
IN_DIM = 16
OUT_DIM = 16
E_DIM = 6
EDGE_H = 64
AUG = 32

NW = 32
EDGES_PER_W = E_PAD // NW
G_EDGES = 1024
N_GROUPS = EDGES_PER_W // G_EDGES
CH = 128
NCH = EDGES_PER_W // CH
N_PAD = 10240
ROWS_PER_TILE = N_PAD // 16

HS_ROWS = E_PAD * IN_DIM // 128
HSR_PER_W = HS_ROWS // NW
MSG_ROWS = E_PAD * AUG // 128
MSGR_PER_W = MSG_ROWS // NW

BE = 2048
GRID = E_PAD // BE
RB = BE // 8


@functools.cache
def _sc_mesh():
    return plsc.VectorSubcoreMesh(core_axis_name="c", subcore_axis_name="s",
                                  num_cores=2)


@functools.cache
def _gather_hsrc_kernel(ng):
    @functools.partial(
        pl.kernel,
        mesh=_sc_mesh(),
        compiler_params=pltpu.CompilerParams(use_tc_tiling_on_sc=False),
        out_type=jax.ShapeDtypeStruct((ng * NW * CH, 128), jnp.float32),
        scratch_types=[
            pltpu.VMEM((ng, 8, CH), jnp.int32),
            pltpu.VMEM((2, G_EDGES, IN_DIM), jnp.float32),
            pltpu.SemaphoreType.DMA,
            pltpu.SemaphoreType.DMA,
        ],
    )
    def _gather_hsrc(h_hbm, src_hbm, out_hbm, idx_v, rows_v, sem_g, sem_w):
        c = lax.axis_index("c")
        s = lax.axis_index("s")
        w = c * 16 + s
        for g in range(ng):
            pltpu.sync_copy(src_hbm.at[w + NW * g], idx_v.at[g])

        def fire(g, b):
            for p in range(8):
                pltpu.async_copy(
                    h_hbm.at[idx_v.at[g, p]],
                    rows_v.at[b, pl.ds(p * CH, CH)], sem_g)

        def drain(g, b):
            for p in range(8):
                pltpu.make_async_copy(
                    h_hbm.at[idx_v.at[g, p]],
                    rows_v.at[b, pl.ds(p * CH, CH)], sem_g).wait()

        def wb(g, b, do_wait):
            for p in range(8):
                a = (rows_v.at[b, pl.ds(p * CH, CH)],
                     out_hbm.at[pl.ds((w + NW * g) * CH, CH),
                                pl.ds(p * IN_DIM, IN_DIM)])
                if do_wait:
                    pltpu.make_async_copy(*a, sem_w).wait()
                else:
                    pltpu.async_copy(*a, sem_w)

        fire(0, 0)
        for g in range(ng):
            b = g % 2
            drain(g, b)
            if g >= 2:
                wb(g - 2, b, True)
            if g + 1 < ng:
                fire(g + 1, (g + 1) % 2)
            wb(g, b, False)
        for g in (ng - 2, ng - 1):
            wb(g, g % 2, True)

    return _gather_hsrc


@functools.cache
def _scatter_agg_kernel(ng):
    @functools.partial(
        pl.kernel,
        mesh=_sc_mesh(),
        compiler_params=pltpu.CompilerParams(use_tc_tiling_on_sc=False),
        out_type=jax.ShapeDtypeStruct((2, N_PAD, AUG), jnp.float32),
        scratch_types=[
            pltpu.VMEM((ng, 8, CH), jnp.int32),
            pltpu.VMEM((2, G_EDGES, AUG), jnp.float32),
            pltpu.VMEM_SHARED((N_PAD, AUG), jnp.float32),
            pltpu.SemaphoreType.DMA,
            pltpu.SemaphoreType.DMA,
        ],
    )
    def _scatter(msg_hbm, dst_hbm, zero_hbm, out_hbm, idx_v, rows_v, agg_sh,
                 sem_r, sem_s):
        c = lax.axis_index("c")
        s = lax.axis_index("s")
        w = c * 16 + s
        pltpu.sync_copy(zero_hbm.at[pl.ds(s * ROWS_PER_TILE, ROWS_PER_TILE)],
                        agg_sh.at[pl.ds(s * ROWS_PER_TILE, ROWS_PER_TILE)])
        for g in range(ng):
            pltpu.sync_copy(dst_hbm.at[w + NW * g], idx_v.at[g])
        plsc.subcore_barrier()

        def read(g, b, do_wait):
            for q in range(4):
                a = (msg_hbm.at[pl.ds((w + NW * g) * 256, 256),
                                pl.ds(q * AUG, AUG)],
                     rows_v.at[b, pl.ds(q * 256, 256)])
                if do_wait:
                    pltpu.make_async_copy(*a, sem_r).wait()
                else:
                    pltpu.async_copy(*a, sem_r)

        def scatters(g, b, do_wait):
            for cch in range(8):
                a = (rows_v.at[b, pl.ds(cch * CH, CH)],
                     agg_sh.at[idx_v.at[g, cch]])
                if do_wait:
                    pltpu.make_async_copy(*a, sem_s).wait()
                else:
                    pltpu.async_copy(*a, sem_s, add=True)

        read(0, 0, False)
        for g in range(ng):
            b = g % 2
            read(g, b, True)
            if g >= 2:
                scatters(g - 2, b, True)
            if g + 1 < ng:
                read(g + 1, (g + 1) % 2, False)
            scatters(g, b, False)
        for g in (ng - 2, ng - 1):
            scatters(g, g % 2, True)
        plsc.subcore_barrier()
        pltpu.sync_copy(agg_sh.at[pl.ds(s * ROWS_PER_TILE, ROWS_PER_TILE)],
                        out_hbm.at[c, pl.ds(s * ROWS_PER_TILE, ROWS_PER_TILE)])

    return _scatter


def _dense_body(blk_off, e_ref, hs_ref, w1_ref, b1_ref, w2_ref, b2_ref,
                r_ref, s_ref, out_ref):
    et = e_ref[...]
    hid = jnp.maximum(
        lax.dot_general(et, w1_ref[...], (((0,), (0,)), ((), ())),
                        preferred_element_type=jnp.float32)
        + b1_ref[...], 0.0)
    ew = (jnp.dot(hid, w2_ref[...], preferred_element_type=jnp.float32)
          + b2_ref[...])
    hs128 = hs_ref[...]
    gi = pl.program_id(0) + blk_off
    rowid = lax.broadcasted_iota(jnp.int32, (RB, 1), 0)
    colid = lax.broadcasted_iota(jnp.int32, (RB, AUG - OUT_DIM), 1)
    onepad = jnp.where(colid == 0, 1.0, 0.0)
    augs = []
    for p in range(8):
        hs_p = hs128[:, p * IN_DIM:(p + 1) * IN_DIM]
        ew_p = ew[p * RB:(p + 1) * RB]
        hrep = jnp.dot(hs_p, r_ref[...], preferred_element_type=jnp.float32)
        msg = jnp.dot(hrep * ew_p, s_ref[...],
                      preferred_element_type=jnp.float32)
        aug = jnp.concatenate([msg, onepad], axis=1)
        mask = rowid < (N_EDGES - BE * gi - RB * p)
        augs.append(jnp.where(mask, aug, 0.0))
    top = jnp.concatenate(augs[0:4], axis=1)
    bot = jnp.concatenate(augs[4:8], axis=1)
    out_ref[...] = jnp.concatenate([top, bot], axis=0)


def _dense_msg(blk_off, nblk, e_t, hs_pk, W1, b1, W2, b2, R, S):
    boff = blk_off
    return pl.pallas_call(
        functools.partial(_dense_body, blk_off),
        grid=(nblk,),
        in_specs=[
            pl.BlockSpec((E_DIM, BE), lambda i: (0, i + boff)),
            pl.BlockSpec((RB, 128), lambda i: (i, 0)),
            pl.BlockSpec((E_DIM, EDGE_H), lambda i: (0, 0)),
            pl.BlockSpec((1, EDGE_H), lambda i: (0, 0)),
            pl.BlockSpec((EDGE_H, IN_DIM * OUT_DIM), lambda i: (0, 0)),
            pl.BlockSpec((1, IN_DIM * OUT_DIM), lambda i: (0, 0)),
            pl.BlockSpec((IN_DIM, IN_DIM * OUT_DIM), lambda i: (0, 0)),
            pl.BlockSpec((IN_DIM * OUT_DIM, OUT_DIM), lambda i: (0, 0)),
        ],
        out_specs=pl.BlockSpec((2 * RB, 128), lambda i: (i, 0)),
        out_shape=jax.ShapeDtypeStruct((nblk * 2 * RB, 128), jnp.float32),
    )(e_t, hs_pk, W1, b1, W2, b2, R, S)


def _final_body(pa_ref, pb_ref, bias_ref, gamma_ref, beta_ref, out_ref):
    p = (pa_ref[0, 0:N_NODES] + pa_ref[1, 0:N_NODES]
         + pb_ref[0, 0:N_NODES] + pb_ref[1, 0:N_NODES])
    agg = p[:, 0:OUT_DIM]
    deg = p[:, OUT_DIM:OUT_DIM + 1]
    rst = agg / jnp.maximum(deg, 1.0) + bias_ref[...]
    rst = jnp.maximum(rst, 0.0)
    mean = jnp.mean(rst, axis=0, keepdims=True)
    var = jnp.mean((rst - mean) * (rst - mean), axis=0, keepdims=True)
    out_ref[...] = ((rst - mean) * lax.rsqrt(var + 1e-5) * gamma_ref[...]
                    + beta_ref[...])


def _finalize(parts_a, parts_b, nn_bias, gamma, beta):
    return pl.pallas_call(
        _final_body,
        in_specs=[
            pl.BlockSpec((2, N_PAD, AUG), lambda: (0, 0, 0)),
            pl.BlockSpec((2, N_PAD, AUG), lambda: (0, 0, 0)),
            pl.BlockSpec((1, OUT_DIM), lambda: (0, 0)),
            pl.BlockSpec((1, OUT_DIM), lambda: (0, 0)),
            pl.BlockSpec((1, OUT_DIM), lambda: (0, 0)),
        ],
        out_specs=pl.BlockSpec((N_NODES, OUT_DIM), lambda: (0, 0)),
        out_shape=jax.ShapeDtypeStruct((N_NODES, OUT_DIM), jnp.float32),
    )(parts_a, parts_b, nn_bias, gamma, beta)


def kernel(h, e, edge_index, W1, b1, W2, b2, nn_bias, gamma, beta):
    pad = E_PAD - N_EDGES
    src = jnp.pad(edge_index[0], (0, pad))
    dst = jnp.pad(edge_index[1], (0, pad))
    e_t = jnp.pad(e.T, ((0, 0), (0, pad)))

    src_sc = (src.reshape(GRID, 8, 2, CH)
              .transpose(0, 2, 1, 3).reshape(2 * GRID, 8, CH))
    dst_sc = dst.reshape(2 * GRID, 8, CH)

    eye = jnp.eye(OUT_DIM, dtype=jnp.float32)
    R = jnp.kron(jnp.eye(IN_DIM, dtype=jnp.float32),
                 jnp.ones((1, OUT_DIM), jnp.float32))
    S = jnp.tile(eye, (IN_DIM, 1))
    zero = jnp.zeros((N_PAD, AUG), jnp.float32)
    b1r = b1.reshape(1, EDGE_H)
    b2r = b2.reshape(1, IN_DIM * OUT_DIM)

    NGA = 2
    nblk_a = NGA * NW // 2
    hs_a = _gather_hsrc_kernel(NGA)(h, src_sc[0:NGA * NW])
    hs_b = _gather_hsrc_kernel(N_GROUPS - NGA)(h, src_sc[NGA * NW:])
    msg_a = _dense_msg(0, nblk_a, e_t, hs_a, W1, b1r, W2, b2r, R, S)
    msg_b = _dense_msg(nblk_a, GRID - nblk_a, e_t, hs_b, W1, b1r, W2, b2r,
                       R, S)
    parts_a = _scatter_agg_kernel(NGA)(msg_a, dst_sc[0:NGA * NW], zero)
    parts_b = _scatter_agg_kernel(N_GROUPS - NGA)(msg_b, dst_sc[NGA * NW:],
                                                  zero)

    return _finalize(parts_a, parts_b, nn_bias.reshape(1, OUT_DIM),
                     gamma.reshape(1, OUT_DIM), beta.reshape(1, OUT_DIM))
